# Initial kernel scaffold; baseline (speedup 1.0000x reference)
import jax
import jax.numpy as jnp
from jax.experimental import pallas as pl


def kernel(decimal_tensor, B):
    raise NotImplementedError("write your pallas kernel here")


# speedup vs baseline: 3.5494x; 3.5494x over previous
import functools

import jax
import jax.numpy as jnp
from jax import lax
from jax.experimental import pallas as pl
from jax.experimental.pallas import tpu as pltpu
from jax.experimental.pallas import tpu_sc as plsc

_R, _S = 4096, 200
_D = 64
_B = _R * _S
_NC, _NS = 2, 16
_NW = _NC * _NS
_BPW = _B // _NW
_CHUNK = 128
_NCHUNK = _BPW // _CHUNK

_mesh = plsc.VectorSubcoreMesh(core_axis_name="c", subcore_axis_name="s")


@functools.partial(
    pl.kernel,
    mesh=_mesh,
    out_type=jax.ShapeDtypeStruct((_B, _D), jnp.float32),
    scratch_types=[
        pltpu.VMEM((_BPW,), jnp.int32),
        pltpu.VMEM((_CHUNK, _D), jnp.float32),
        pltpu.SemaphoreType.DMA,
    ],
    compiler_params=pltpu.CompilerParams(use_tc_tiling_on_sc=False),
)
def _gather_sc(idx_hbm, table_hbm, out_hbm, idx_v, rows_v, gsem):
    wid = lax.axis_index("s") * _NC + lax.axis_index("c")
    base = wid * _BPW
    pltpu.sync_copy(idx_hbm.at[pl.ds(base, _BPW)], idx_v)

    def step(j, carry):
        off = j * _CHUNK
        pltpu.async_copy(
            table_hbm.at[idx_v.at[pl.ds(off, _CHUNK)]], rows_v, gsem
        ).wait()
        pltpu.sync_copy(rows_v, out_hbm.at[pl.ds(base + off, _CHUNK)])
        return carry

    lax.fori_loop(0, _NCHUNK, step, 0)


def kernel(decimal_tensor, B):
    idx = decimal_tensor.reshape(-1).astype(jnp.int32)
    out = _gather_sc(idx, B)
    return out.reshape(_R, _S, _D)


# speedup vs baseline: 4.2664x; 1.2020x over previous
import functools

import jax
import jax.numpy as jnp
from jax import lax
from jax.experimental import pallas as pl
from jax.experimental.pallas import tpu as pltpu
from jax.experimental.pallas import tpu_sc as plsc

_R, _S = 4096, 200
_D = 64
_B = _R * _S
_NC, _NS = 2, 16
_NW = _NC * _NS
_BPW = _B // _NW
_CHUNK = 128
_K = 2
_GROW = _K * _CHUNK
_NG = _BPW // _GROW
_NBUF = 4
_NROUND = _NG // _NBUF

_mesh = plsc.VectorSubcoreMesh(core_axis_name="c", subcore_axis_name="s")


@functools.partial(
    pl.kernel,
    mesh=_mesh,
    out_type=jax.ShapeDtypeStruct((_B, _D), jnp.float32),
    scratch_types=[
        pltpu.VMEM((_BPW,), jnp.int32),
        [pltpu.VMEM((_GROW, _D), jnp.float32)] * _NBUF,
        [pltpu.SemaphoreType.DMA] * _NBUF,
        [pltpu.SemaphoreType.DMA] * _NBUF,
    ],
    compiler_params=pltpu.CompilerParams(use_tc_tiling_on_sc=False),
)
def _gather_sc(idx_hbm, table_hbm, out_hbm, idx_v, bufs, gsems, osems):
    wid = lax.axis_index("s") * _NC + lax.axis_index("c")
    base = wid * _BPW
    pltpu.sync_copy(idx_hbm.at[pl.ds(base, _BPW)], idx_v)

    def fire_g(bi, grp):
        for b in range(_K):
            off = grp * _GROW + b * _CHUNK
            pltpu.async_copy(
                table_hbm.at[idx_v.at[pl.ds(off, _CHUNK)]],
                bufs[bi].at[pl.ds(b * _CHUNK, _CHUNK)],
                gsems[bi],
            )

    def wait_g(bi):
        pltpu.make_async_copy(
            out_hbm.at[pl.ds(0, _GROW)], bufs[bi], gsems[bi]
        ).wait()

    def fire_o(bi, grp):
        pltpu.async_copy(
            bufs[bi], out_hbm.at[pl.ds(base + grp * _GROW, _GROW)], osems[bi]
        )

    def wait_o(bi):
        pltpu.make_async_copy(
            bufs[bi], out_hbm.at[pl.ds(0, _GROW)], osems[bi]
        ).wait()

    fire_g(0, 0)
    fire_g(1, 1)

    wait_g(0); fire_o(0, 0); fire_g(2, 2)
    wait_g(1); fire_o(1, 1); fire_g(3, 3)
    wait_g(2); fire_o(2, 2); wait_o(0); fire_g(0, 4)
    wait_g(3); fire_o(3, 3); wait_o(1); fire_g(1, 5)

    def round_body(r, carry):
        for bi in range(_NBUF):
            grp = r * _NBUF + bi
            pj = (bi + 2) % _NBUF
            wait_g(bi)
            fire_o(bi, grp)
            wait_o(pj)
            fire_g(pj, grp + 2)
        return carry

    lax.fori_loop(1, _NROUND - 1, round_body, 0)

    r = _NROUND - 1
    wait_g(0); fire_o(0, r * _NBUF + 0); wait_o(2); fire_g(2, r * _NBUF + 2)
    wait_g(1); fire_o(1, r * _NBUF + 1); wait_o(3); fire_g(3, r * _NBUF + 3)
    wait_g(2); fire_o(2, r * _NBUF + 2)
    wait_g(3); fire_o(3, r * _NBUF + 3)
    for bi in range(_NBUF):
        wait_o(bi)


def kernel(decimal_tensor, B):
    idx = decimal_tensor.reshape(-1).astype(jnp.int32)
    out = _gather_sc(idx, B)
    return out.reshape(_R, _S, _D)
